# Initial kernel scaffold; baseline (speedup 1.0000x reference)
import jax
import jax.numpy as jnp
from jax.experimental import pallas as pl


def kernel(x, edge_index, batch, bn_feat_g, bn_feat_b, Wf, bf, bns_g, bns_b, Ws, bs, bnfc_g, bnfc_b, Wl, bl, bnh_g, bnh_b, Wc, bc):
    raise NotImplementedError("write your pallas kernel here")


# speedup vs baseline: 11.7412x; 11.7412x over previous
import functools

import jax
import jax.numpy as jnp
from jax import lax
from jax.experimental import pallas as pl
from jax.experimental.pallas import tpu as pltpu
from jax.experimental.pallas import tpu_sc as plsc

N, E, D, H, G = 10000, 320000, 128, 128, 64
NC, NS, L = 2, 16, 16
NW = NC * NS
EPT = E // NW
CHUNK = 128
NCHUNK = EPT // CHUNK
TAIL = EPT - NCHUNK * CHUNK
NP = 10240
RPT = NP // NS
ZR = 128


def _mesh():
    return plsc.VectorSubcoreMesh(
        core_axis_name="c", subcore_axis_name="s", num_cores=NC, num_subcores=NS
    )


def _fill_zeros(ref, rows, cols):
    vec = jnp.zeros((L,), jnp.float32)

    def body(i, carry):
        for j in range(cols // L):
            ref[i, pl.ds(j * L, L)] = vec
        return carry

    lax.fori_loop(0, rows, body, 0)


@functools.cache
def _sc_degree_kernel():
    return pl.kernel(
        _sc_degree_body,
        out_type=jax.ShapeDtypeStruct((NC, NP, H), jnp.float32),
        mesh=_mesh(),
        scratch_types=[
            pltpu.VMEM((CHUNK,), jnp.int32),
            pltpu.VMEM((TAIL,), jnp.int32),
            pltpu.VMEM((CHUNK, H), jnp.float32),
            pltpu.VMEM((ZR, H), jnp.float32),
            pltpu.VMEM_SHARED((NP, H), jnp.float32),
        ],
    )


def _sc_degree(dst):
    return _sc_degree_kernel()(dst)


def _sc_degree_body(dst_hbm, out_hbm, dstv, dstt, ones_v, zb, acc):
    cid = lax.axis_index("c")
    sid = lax.axis_index("s")
    wid = sid * NC + cid

    one = jnp.ones((L,), jnp.float32)

    def fill_ones(i, carry):
        for j in range(H // L):
            ones_v[i, pl.ds(j * L, L)] = one
        return carry

    lax.fori_loop(0, CHUNK, fill_ones, 0)
    _fill_zeros(zb, ZR, H)
    for i in range(RPT // ZR):
        pltpu.sync_copy(zb, acc.at[pl.ds(sid * RPT + i * ZR, ZR)])
    plsc.subcore_barrier()

    base = wid * EPT

    def chunk(c, carry):
        pltpu.sync_copy(dst_hbm.at[pl.ds(base + c * CHUNK, CHUNK)], dstv)
        pltpu.sync_copy(ones_v, acc.at[dstv], add=True)
        return carry

    lax.fori_loop(0, NCHUNK, chunk, 0)
    pltpu.sync_copy(dst_hbm.at[pl.ds(base + NCHUNK * CHUNK, TAIL)], dstt)
    pltpu.sync_copy(ones_v.at[pl.ds(0, TAIL)], acc.at[dstt], add=True)

    plsc.subcore_barrier()
    pltpu.sync_copy(
        acc.at[pl.ds(sid * RPT, RPT)], out_hbm.at[cid, pl.ds(sid * RPT, RPT)]
    )


@functools.cache
def _sc_propagate_kernel():
    return pl.kernel(
        _sc_propagate_body,
        out_type=jax.ShapeDtypeStruct((NC, NP, H), jnp.float32),
        mesh=_mesh(),
        scratch_types=[
            pltpu.VMEM((CHUNK,), jnp.int32),
            pltpu.VMEM((CHUNK,), jnp.int32),
            pltpu.VMEM((TAIL,), jnp.int32),
            pltpu.VMEM((TAIL,), jnp.int32),
            pltpu.VMEM((CHUNK, H), jnp.float32),
            pltpu.VMEM((TAIL, H), jnp.float32),
            pltpu.VMEM((ZR, H), jnp.float32),
            pltpu.VMEM_SHARED((NP, H), jnp.float32),
            pltpu.SemaphoreType.DMA,
        ],
    )


def _sc_propagate(g, src, dst):
    return _sc_propagate_kernel()(g, src, dst)


def _sc_propagate_body(g_hbm, src_hbm, dst_hbm, out_hbm,
                       srcv, dstv, srct, dstt, rows, rowst, zb, acc, sem):
    cid = lax.axis_index("c")
    sid = lax.axis_index("s")
    wid = sid * NC + cid

    _fill_zeros(zb, ZR, H)
    for i in range(RPT // ZR):
        pltpu.sync_copy(zb, acc.at[pl.ds(sid * RPT + i * ZR, ZR)])
    plsc.subcore_barrier()

    base = wid * EPT

    def chunk(c, carry):
        off = base + c * CHUNK
        pltpu.sync_copy(src_hbm.at[pl.ds(off, CHUNK)], srcv)
        pltpu.sync_copy(dst_hbm.at[pl.ds(off, CHUNK)], dstv)
        pltpu.async_copy(g_hbm.at[srcv], rows, sem).wait()
        pltpu.sync_copy(rows, acc.at[dstv], add=True)
        return carry

    lax.fori_loop(0, NCHUNK, chunk, 0)
    off = base + NCHUNK * CHUNK
    pltpu.sync_copy(src_hbm.at[pl.ds(off, TAIL)], srct)
    pltpu.sync_copy(dst_hbm.at[pl.ds(off, TAIL)], dstt)
    pltpu.async_copy(g_hbm.at[srct], rowst, sem).wait()
    pltpu.sync_copy(rowst, acc.at[dstt], add=True)

    plsc.subcore_barrier()
    pltpu.sync_copy(
        acc.at[pl.ds(sid * RPT, RPT)], out_hbm.at[cid, pl.ds(sid * RPT, RPT)]
    )


def _bn(h, g2, b2):
    m = jnp.mean(h, axis=0, keepdims=True)
    d = h - m
    v = jnp.mean(d * d, axis=0, keepdims=True)
    return d * lax.rsqrt(v + 1e-5) * g2 + b2


def _dot(a, b):
    return jax.lax.dot_general(
        a, b, (((1,), (0,)), ((), ())),
        preferred_element_type=jnp.float32,
        precision=lax.Precision.HIGHEST,
    )


def _tc_head_body(x_ref, degp_ref, g_ref, b_ref, W_ref, bias_ref,
                  out_ref, dinv_ref):
    deg = 1.0 + degp_ref[0, :N, 0:1] + degp_ref[1, :N, 0:1]
    dinv = lax.rsqrt(deg)
    dinv_ref[...] = dinv
    h = _bn(x_ref[...], g_ref[...].reshape(1, D), b_ref[...].reshape(1, D))
    out_ref[...] = dinv * (_dot(h, W_ref[...]) + bias_ref[...].reshape(1, H))


def _tc_head(x, degp, g, b, W, bias):
    return pl.pallas_call(
        _tc_head_body,
        out_shape=(
            jax.ShapeDtypeStruct((N, H), jnp.float32),
            jax.ShapeDtypeStruct((N, 1), jnp.float32),
        ),
    )(x, degp, g, b, W, bias)


def _tc_mid_body(p_ref, gp_ref, dinv_ref, g_ref, b_ref, W_ref, bias_ref,
                 out_ref):
    dinv = dinv_ref[...]
    h = jax.nn.relu(dinv * (p_ref[0, :N] + p_ref[1, :N] + gp_ref[...]))
    h = _bn(h, g_ref[...].reshape(1, H), b_ref[...].reshape(1, H))
    out_ref[...] = dinv * (_dot(h, W_ref[...]) + bias_ref[...].reshape(1, H))


def _tc_mid(p, gprev, dinv, g, b, W, bias):
    return pl.pallas_call(
        _tc_mid_body,
        out_shape=jax.ShapeDtypeStruct((N, H), jnp.float32),
    )(p, gprev, dinv, g, b, W, bias)


def _tc_tail_body(p_ref, gp_ref, dinv_ref, batch_ref,
                  bnfc_g_ref, bnfc_b_ref, Wl_ref, bl_ref,
                  bnh_g_ref, bnh_b_ref, Wc_ref, bc_ref, out_ref):
    h = jax.nn.relu(dinv_ref[...] * (p_ref[0, :N] + p_ref[1, :N] + gp_ref[...]))
    gid = lax.broadcasted_iota(jnp.int32, (G, N), 0)
    onehot_t = (gid == batch_ref[...].reshape(1, N)).astype(jnp.float32)
    pooled = _dot(onehot_t, h)
    z = _bn(pooled, bnfc_g_ref[...].reshape(1, H), bnfc_b_ref[...].reshape(1, H))
    z = jax.nn.relu(_dot(z, Wl_ref[...]) + bl_ref[...].reshape(1, H))
    z = _bn(z, bnh_g_ref[...].reshape(1, H), bnh_b_ref[...].reshape(1, H))
    out_ref[...] = jax.nn.sigmoid(_dot(z, Wc_ref[...]) + bc_ref[...].reshape(1, 1))


def _tc_tail(p, gprev, dinv, batch, bnfc_g, bnfc_b, Wl, bl, bnh_g, bnh_b,
             Wc, bc):
    return pl.pallas_call(
        _tc_tail_body,
        out_shape=jax.ShapeDtypeStruct((G, 1), jnp.float32),
    )(p, gprev, dinv, batch, bnfc_g, bnfc_b, Wl, bl, bnh_g, bnh_b, Wc, bc)


def kernel(x, edge_index, batch, bn_feat_g, bn_feat_b, Wf, bf, bns_g, bns_b,
           Ws, bs, bnfc_g, bnfc_b, Wl, bl, bnh_g, bnh_b, Wc, bc):
    src = edge_index[0]
    dst = edge_index[1]

    degp = _sc_degree(dst)
    g, dinv = _tc_head(x, degp, bn_feat_g, bn_feat_b, Wf, bf)
    for i in range(3):
        p = _sc_propagate(g, src, dst)
        g = _tc_mid(p, g, dinv, bns_g[i], bns_b[i], Ws[i], bs[i])
    p = _sc_propagate(g, src, dst)
    out = _tc_tail(p, g, dinv, batch, bnfc_g, bnfc_b, Wl, bl,
                   bnh_g, bnh_b, Wc, bc)
    return out.reshape(-1)
